# Initial kernel scaffold; baseline (speedup 1.0000x reference)
import jax
import jax.numpy as jnp
from jax.experimental import pallas as pl


def kernel(nodes, x, table, alpha):
    raise NotImplementedError("write your pallas kernel here")


# speedup vs baseline: 1.6412x; 1.6412x over previous
import jax
import jax.numpy as jnp
from jax import lax
from jax.experimental import pallas as pl
from jax.experimental.pallas import tpu as pltpu
from jax.experimental.pallas import tpu_sc as plsc

N = 49688
D = 128
B = 8
P = 512
E = B * P
NW = 32
EPW = E // NW
RBLK = 1024
NB = -(-N // RBLK)
L = 16


def _dense_body(tbl_ref, nodes_ref, nrow_ref, alpha_ref, out_ref, xsrc_ref,
                ag_ref):
    i = pl.program_id(0)
    b = pl.program_id(1)
    out_ref[...] = tbl_ref[...][None]

    @pl.when((i == 0) & (b == 0))
    def _():
        q_iota = lax.broadcasted_iota(jnp.int32, (P, P), 1)
        for bb in range(B):
            row = nodes_ref[bb, :]
            eq = row[:, None] == row[None, :]
            lastq = jnp.max(jnp.where(eq, q_iota, -1), axis=1)
            xsrc_ref[bb, :] = lastq + bb * P

    rows = lax.broadcasted_iota(jnp.int32, (RBLK, 1), 0) + i * RBLK
    nrow = nrow_ref[0, 0, :]
    eq = nrow[None, :] == rows
    contrib = jnp.sum(jnp.where(eq, alpha_ref[...], 0.0), axis=0)

    @pl.when(i == 0)
    def _():
        ag_ref[pl.ds(b, 1), :] = contrib[None]

    @pl.when(i > 0)
    def _():
        ag_ref[pl.ds(b, 1), :] += contrib[None]


def _dense_copy(table, nodes, alpha):
    return pl.pallas_call(
        _dense_body,
        grid=(NB, B),
        in_specs=[
            pl.BlockSpec((RBLK, D), lambda i, b: (i, 0)),
            pl.BlockSpec((B, P), lambda i, b: (0, 0)),
            pl.BlockSpec((1, 1, P), lambda i, b: (b, 0, 0)),
            pl.BlockSpec((RBLK, 1), lambda i, b: (i, 0)),
        ],
        out_specs=[
            pl.BlockSpec((1, RBLK, D), lambda i, b: (b, i, 0)),
            pl.BlockSpec((B, P), lambda i, b: (0, 0)),
            pl.BlockSpec((B, P), lambda i, b: (0, 0)),
        ],
        out_shape=[
            jax.ShapeDtypeStruct((B, N, D), jnp.float32),
            jax.ShapeDtypeStruct((B, P), jnp.int32),
            jax.ShapeDtypeStruct((B, P), jnp.float32),
        ],
    )(table, nodes, nodes.reshape(B, 1, P), alpha)


def _sc_update_body(out_hbm, nodes_hbm, xsrc_hbm, ag_hbm, x_hbm, table_hbm,
                    idx_v, xsrc_v, fidx_v, tbl_v, x_v, a_v, sem1, sem2):
    c = lax.axis_index("c")
    s = lax.axis_index("s")
    wid = s * 2 + c
    base = wid * EPW
    pltpu.sync_copy(nodes_hbm.at[pl.ds(base, EPW)], idx_v)
    pltpu.sync_copy(xsrc_hbm.at[pl.ds(base, EPW)], xsrc_v)
    pltpu.sync_copy(ag_hbm.at[pl.ds(base, EPW)], a_v)
    cp1 = pltpu.async_copy(table_hbm.at[idx_v], tbl_v, sem1)
    cp2 = pltpu.async_copy(x_hbm.at[xsrc_v], x_v, sem2)
    boff = (base // P) * N
    for j in range(EPW // L):
        fidx_v[pl.ds(j * L, L)] = idx_v[pl.ds(j * L, L)] + boff
    cp1.wait()
    cp2.wait()

    def row_body(i, carry):
        a = plsc.load_gather(a_v, [jnp.full((L,), i, jnp.int32)])
        for j in range(D // L):
            sl = pl.ds(j * L, L)
            t = tbl_v[i, sl]
            xx = x_v[i, sl]
            tbl_v[i, sl] = t + a * (xx - t)
        return carry

    lax.fori_loop(0, EPW, row_body, 0)
    pltpu.async_copy(tbl_v, out_hbm.at[fidx_v], sem1).wait()


_sc_update = pl.kernel(
    _sc_update_body,
    out_type=(),
    mesh=plsc.VectorSubcoreMesh(
        core_axis_name="c", subcore_axis_name="s", num_cores=2,
        num_subcores=16),
    compiler_params=pltpu.CompilerParams(needs_layout_passes=False),
    scratch_types=[
        pltpu.VMEM((EPW,), jnp.int32),
        pltpu.VMEM((EPW,), jnp.int32),
        pltpu.VMEM((EPW,), jnp.int32),
        pltpu.VMEM((EPW, D), jnp.float32),
        pltpu.VMEM((EPW, D), jnp.float32),
        pltpu.VMEM((EPW,), jnp.float32),
        pltpu.SemaphoreType.DMA,
        pltpu.SemaphoreType.DMA,
    ],
)


def kernel(nodes, x, table, alpha):
    dense, xsrc, ag = _dense_copy(table, nodes, alpha)
    out_ref = jax.new_ref(dense.reshape(B * N, D))
    _sc_update(out_ref, nodes.reshape(E), xsrc.reshape(E), ag.reshape(E), x,
               table)
    return out_ref[...].reshape(B, N, D)


# speedup vs baseline: 4.9276x; 3.0025x over previous
import jax
import jax.numpy as jnp
from jax import lax
from jax.experimental import pallas as pl
from jax.experimental.pallas import tpu as pltpu
from jax.experimental.pallas import tpu_sc as plsc

N = 49688
D = 128
B = 8
P = 512
E = B * P
NW = 32
EPW = E // NW
RBLK = 1024
NB = -(-N // RBLK)
NPAD = NB * RBLK
L = 16


def _dense_body(tbl_ref, nodes_ref, out_ref, xsrc_ref):
    i = pl.program_id(0)
    out_ref[...] = jnp.broadcast_to(tbl_ref[...][None], (B, RBLK, D))

    @pl.when(i == 0)
    def _():
        q_iota = lax.broadcasted_iota(jnp.int32, (P, P), 1)
        for bb in range(B):
            row = nodes_ref[bb, :]
            eq = row[:, None] == row[None, :]
            lastq = jnp.max(jnp.where(eq, q_iota, -1), axis=1)
            xsrc_ref[bb, :] = lastq + bb * P


def _dense_copy(table, nodes):
    return pl.pallas_call(
        _dense_body,
        grid=(NB,),
        in_specs=[
            pl.BlockSpec((RBLK, D), lambda i: (i, 0)),
            pl.BlockSpec((B, P), lambda i: (0, 0)),
        ],
        out_specs=[
            pl.BlockSpec((B, RBLK, D), lambda i: (0, i, 0)),
            pl.BlockSpec((B, P), lambda i: (0, 0)),
        ],
        out_shape=[
            jax.ShapeDtypeStruct((B, N, D), jnp.float32),
            jax.ShapeDtypeStruct((B, P), jnp.int32),
        ],
    )(table, nodes)


def _sc_update_body(out_hbm, nodes_hbm, xsrc_hbm, x_hbm, table_hbm, alpha_hbm,
                    idx_v, xsrc_v, fidx_v, aidx_v, tbl_v, x_v, ar_v, a_v,
                    sem1, sem2, sem3):
    c = lax.axis_index("c")
    s = lax.axis_index("s")
    wid = s * 2 + c
    base = wid * EPW
    pltpu.sync_copy(nodes_hbm.at[pl.ds(base, EPW)], idx_v)
    pltpu.sync_copy(xsrc_hbm.at[pl.ds(base, EPW)], xsrc_v)
    boff = (base // P) * N
    for j in range(EPW // L):
        sl = pl.ds(j * L, L)
        v = idx_v[sl]
        fidx_v[sl] = v + boff
        aidx_v[sl] = lax.shift_right_logical(v, 7)
    cp1 = pltpu.async_copy(table_hbm.at[idx_v], tbl_v, sem1)
    cp2 = pltpu.async_copy(x_hbm.at[xsrc_v], x_v, sem2)
    cp3 = pltpu.async_copy(alpha_hbm.at[aidx_v], ar_v, sem3)
    cp3.wait()
    for g in range(EPW // L):
        sl = pl.ds(g * L, L)
        ent = lax.iota(jnp.int32, L) + g * L
        cols = idx_v[sl] & 127
        a_v[sl] = plsc.load_gather(ar_v, [ent, cols])
    cp1.wait()
    cp2.wait()

    def row_body(i, carry):
        a = plsc.load_gather(a_v, [jnp.full((L,), i, jnp.int32)])
        for j in range(D // L):
            sl = pl.ds(j * L, L)
            t = tbl_v[i, sl]
            xx = x_v[i, sl]
            tbl_v[i, sl] = t + a * (xx - t)
        return carry

    lax.fori_loop(0, EPW, row_body, 0)
    pltpu.async_copy(tbl_v, out_hbm.at[fidx_v], sem1).wait()


_sc_update = pl.kernel(
    _sc_update_body,
    out_type=(),
    mesh=plsc.VectorSubcoreMesh(
        core_axis_name="c", subcore_axis_name="s", num_cores=2,
        num_subcores=16),
    compiler_params=pltpu.CompilerParams(needs_layout_passes=False),
    scratch_types=[
        pltpu.VMEM((EPW,), jnp.int32),
        pltpu.VMEM((EPW,), jnp.int32),
        pltpu.VMEM((EPW,), jnp.int32),
        pltpu.VMEM((EPW,), jnp.int32),
        pltpu.VMEM((EPW, D), jnp.float32),
        pltpu.VMEM((EPW, D), jnp.float32),
        pltpu.VMEM((EPW, D), jnp.float32),
        pltpu.VMEM((EPW,), jnp.float32),
        pltpu.SemaphoreType.DMA,
        pltpu.SemaphoreType.DMA,
        pltpu.SemaphoreType.DMA,
    ],
)


def kernel(nodes, x, table, alpha):
    dense, xsrc = _dense_copy(table, nodes)
    alpha2d = jnp.pad(alpha.reshape(N), (0, NPAD - N)).reshape(NPAD // D, D)
    out_ref = jax.new_ref(dense.reshape(B * N, D))
    _sc_update(out_ref, nodes.reshape(E), xsrc.reshape(E), x, table, alpha2d)
    return out_ref[...].reshape(B, N, D)


# speedup vs baseline: 5.1352x; 1.0421x over previous
import jax
import jax.numpy as jnp
from jax import lax
from jax.experimental import pallas as pl
from jax.experimental.pallas import tpu as pltpu
from jax.experimental.pallas import tpu_sc as plsc

N = 49688
D = 128
B = 8
P = 512
E = B * P
NW = 32
EPW = E // NW
RBLK = 2048
NB = -(-N // RBLK)
NPAD = NB * RBLK
L = 16


def _dense_body(tbl_ref, nodes_ref, out_ref, xsrc_ref):
    i = pl.program_id(0)
    out_ref[...] = jnp.broadcast_to(tbl_ref[...][None], (B, RBLK, D))

    @pl.when(i == 0)
    def _():
        q_iota = lax.broadcasted_iota(jnp.int32, (P, P), 1)
        for bb in range(B):
            row = nodes_ref[bb, :]
            eq = row[:, None] == row[None, :]
            lastq = jnp.max(jnp.where(eq, q_iota, -1), axis=1)
            xsrc_ref[bb, :] = lastq + bb * P


def _dense_copy(table, nodes):
    return pl.pallas_call(
        _dense_body,
        grid=(NB,),
        in_specs=[
            pl.BlockSpec((RBLK, D), lambda i: (i, 0)),
            pl.BlockSpec((B, P), lambda i: (0, 0)),
        ],
        out_specs=[
            pl.BlockSpec((B, RBLK, D), lambda i: (0, i, 0)),
            pl.BlockSpec((B, P), lambda i: (0, 0)),
        ],
        out_shape=[
            jax.ShapeDtypeStruct((B, N, D), jnp.float32),
            jax.ShapeDtypeStruct((B, P), jnp.int32),
        ],
    )(table, nodes)


def _sc_update_body(out_hbm, nodes_hbm, xsrc_hbm, x_hbm, table_hbm, alpha_hbm,
                    idx_v, xsrc_v, fidx_v, aidx_v, tbl_v, x_v, ar_v, a_v,
                    sem1, sem2, sem3):
    c = lax.axis_index("c")
    s = lax.axis_index("s")
    wid = s * 2 + c
    base = wid * EPW
    pltpu.sync_copy(nodes_hbm.at[pl.ds(base, EPW)], idx_v)
    pltpu.sync_copy(xsrc_hbm.at[pl.ds(base, EPW)], xsrc_v)
    boff = (base // P) * N
    for j in range(EPW // L):
        sl = pl.ds(j * L, L)
        v = idx_v[sl]
        fidx_v[sl] = v + boff
        aidx_v[sl] = lax.shift_right_logical(v, 7)
    cp1 = pltpu.async_copy(table_hbm.at[idx_v], tbl_v, sem1)
    cp2 = pltpu.async_copy(x_hbm.at[xsrc_v], x_v, sem2)
    cp3 = pltpu.async_copy(alpha_hbm.at[aidx_v], ar_v, sem3)
    cp3.wait()
    for g in range(EPW // L):
        sl = pl.ds(g * L, L)
        ent = lax.iota(jnp.int32, L) + g * L
        cols = idx_v[sl] & 127
        a_v[sl] = plsc.load_gather(ar_v, [ent, cols])
    cp1.wait()
    cp2.wait()

    def row_body(i, carry):
        a = plsc.load_gather(a_v, [jnp.full((L,), i, jnp.int32)])
        for j in range(D // L):
            sl = pl.ds(j * L, L)
            t = tbl_v[i, sl]
            xx = x_v[i, sl]
            tbl_v[i, sl] = t + a * (xx - t)
        return carry

    lax.fori_loop(0, EPW, row_body, 0)
    pltpu.async_copy(tbl_v, out_hbm.at[fidx_v], sem1).wait()


_sc_update = pl.kernel(
    _sc_update_body,
    out_type=(),
    mesh=plsc.VectorSubcoreMesh(
        core_axis_name="c", subcore_axis_name="s", num_cores=2,
        num_subcores=16),
    compiler_params=pltpu.CompilerParams(needs_layout_passes=False),
    scratch_types=[
        pltpu.VMEM((EPW,), jnp.int32),
        pltpu.VMEM((EPW,), jnp.int32),
        pltpu.VMEM((EPW,), jnp.int32),
        pltpu.VMEM((EPW,), jnp.int32),
        pltpu.VMEM((EPW, D), jnp.float32),
        pltpu.VMEM((EPW, D), jnp.float32),
        pltpu.VMEM((EPW, D), jnp.float32),
        pltpu.VMEM((EPW,), jnp.float32),
        pltpu.SemaphoreType.DMA,
        pltpu.SemaphoreType.DMA,
        pltpu.SemaphoreType.DMA,
    ],
)


def kernel(nodes, x, table, alpha):
    dense, xsrc = _dense_copy(table, nodes)
    alpha2d = jnp.pad(alpha.reshape(N), (0, NPAD - N)).reshape(NPAD // D, D)
    out_ref = jax.new_ref(dense.reshape(B * N, D))
    _sc_update(out_ref, nodes.reshape(E), xsrc.reshape(E), x, table, alpha2d)
    return out_ref[...].reshape(B, N, D)
